# baseline (speedup 1.0000x reference)
import jax
import jax.numpy as jnp
from jax.experimental import pallas as pl

_ROWS = 20480


def _copy_kernel(wav_ref, len_ref, wav_out_ref, len_out_ref):
    wav_out_ref[...] = wav_ref[...]
    len_out_ref[...] = len_ref[...]


def kernel(wav_batch, lengths):
    wav2d = wav_batch.reshape(_ROWS, 128)
    lengths_2d = jnp.asarray(lengths).astype(jnp.int32).reshape(1, lengths.shape[0])
    wav_out, len_out = pl.pallas_call(
        _copy_kernel,
        out_shape=[
            jax.ShapeDtypeStruct((_ROWS, 128), wav_batch.dtype),
            jax.ShapeDtypeStruct(lengths_2d.shape, jnp.int32),
        ],
    )(wav2d, lengths_2d)
    return wav_out.reshape(wav_batch.shape), len_out.reshape(lengths.shape)


# speedup vs baseline: 1.4956x; 1.4956x over previous
import functools

import jax
import jax.numpy as jnp
from jax import lax
from jax.experimental import pallas as pl
from jax.experimental.pallas import tpu as pltpu
from jax.experimental.pallas import tpu_sc as plsc

_B, _T, _F = 16, 2048, 80
_HALF_T = _T // 2
_CHUNK_T = 512


@functools.partial(
    pl.kernel,
    out_type=[
        jax.ShapeDtypeStruct((_B, _T, _F), jnp.float32),
        jax.ShapeDtypeStruct((_B,), jnp.int32),
    ],
    mesh=plsc.VectorSubcoreMesh(core_axis_name="c", subcore_axis_name="s"),
    scratch_types=[
        pltpu.VMEM((_CHUNK_T, _F), jnp.float32),
        pltpu.VMEM((_B,), jnp.int32),
    ],
    compiler_params=pltpu.CompilerParams(use_tc_tiling_on_sc=True),
)
def _sc_materialize(wav_hbm, len_hbm, wav_out, len_out, buf, len_buf):
    c = lax.axis_index("c")
    s = lax.axis_index("s")
    wid = s * 2 + c
    b = wid // 2
    t0 = (wid % 2) * _HALF_T
    for k in range(_HALF_T // _CHUNK_T):
        tk = t0 + k * _CHUNK_T
        pltpu.sync_copy(wav_hbm.at[b, pl.ds(tk, _CHUNK_T)], buf)
        pltpu.sync_copy(buf, wav_out.at[b, pl.ds(tk, _CHUNK_T)])

    @pl.when(wid == 0)
    def _():
        pltpu.sync_copy(len_hbm, len_buf)
        pltpu.sync_copy(len_buf, len_out)


def kernel(wav_batch, lengths):
    lengths_i32 = jnp.asarray(lengths).astype(jnp.int32)
    wav_out, len_out = _sc_materialize(wav_batch, lengths_i32)
    return wav_out, len_out
